# baseline (speedup 1.0000x reference)
import functools

import jax
import jax.numpy as jnp
from jax import lax
from jax.experimental import pallas as pl
from jax.experimental.pallas import tpu as pltpu
from jax.experimental.pallas import tpu_sc as plsc

VOCAB = 1000000
EMB = 64
BATCH = 16384
NC = 2
NS = 16
L = 16
SLOTS = BATCH // NS
D_PER_CORE = EMB // NC
NPAIR = D_PER_CORE // 2
NCH = SLOTS // 128
RCH = (VOCAB // NS) // 128 * 128
TAIL_OFF = RCH * NS
TAIL = 640
VPAD = TAIL_OFF + TAIL


def _sc_body(qid_hbm, did_hbm, tabt_hbm, tail_hbm, out_hbm,
             qid_v, did_v, buf0, buf1, qv_v, dv_v, acc_v,
             sem0, sem1, semg):
    c = lax.axis_index("c")
    s = lax.axis_index("s")
    d_base = c * D_PER_CORE

    pltpu.sync_copy(qid_hbm.at[s], qid_v)
    pltpu.sync_copy(did_hbm.at[s], did_v)

    zero = jnp.zeros((L,), jnp.float32)
    for k in range(SLOTS // L):
        acc_v[pl.ds(k * L, L)] = zero

    def row_copies(d, buf, sem):
        sl = pl.ds(s * RCH, RCH)
        tl = pl.ds(TAIL_OFF, TAIL)
        return (pltpu.make_async_copy(tabt_hbm.at[d].at[sl], buf.at[sl], sem),
                pltpu.make_async_copy(tail_hbm.at[d], buf.at[tl], sem))

    def row_start(d, buf, sem):
        main, tail = row_copies(d, buf, sem)
        main.start()

        @pl.when(s == 0)
        def _():
            tail.start()

    def row_wait(d, buf, sem):
        main, tail = row_copies(d, buf, sem)
        main.wait()

        @pl.when(s == 0)
        def _():
            tail.wait()

    def process(buf):
        copies = []
        for j in range(NCH):
            copies.append(pltpu.async_copy(
                buf.at[qid_v.at[j]], qv_v.at[pl.ds(j * 128, 128)], semg))
            copies.append(pltpu.async_copy(
                buf.at[did_v.at[j]], dv_v.at[pl.ds(j * 128, 128)], semg))
        for cp in copies:
            cp.wait()
        for k in range(SLOTS // L):
            sl = pl.ds(k * L, L)
            plsc.addupdate(acc_v.at[sl], qv_v[sl] * dv_v[sl])

    row_start(d_base, buf0, sem0)

    def pair(i, carry):
        r = d_base + i * 2

        row_wait(r, buf0, sem0)
        plsc.subcore_barrier()
        row_start(r + 1, buf1, sem1)
        process(buf0)

        row_wait(r + 1, buf1, sem1)
        plsc.subcore_barrier()

        @pl.when(i + 1 < NPAIR)
        def _():
            row_start(r + 2, buf0, sem0)

        process(buf1)
        return carry

    lax.fori_loop(0, NPAIR, pair, 0)

    pltpu.sync_copy(acc_v, out_hbm.at[c, pl.ds(s * SLOTS, SLOTS)])


@jax.jit
def _run(qid, did, tabt, tail):
    mesh = plsc.VectorSubcoreMesh(core_axis_name="c", subcore_axis_name="s")
    f = functools.partial(
        pl.kernel, mesh=mesh,
        out_type=jax.ShapeDtypeStruct((NC, BATCH), jnp.float32),
        scratch_types=[
            pltpu.VMEM((NCH, 128), jnp.int32),
            pltpu.VMEM((NCH, 128), jnp.int32),
            pltpu.VMEM_SHARED((VPAD,), jnp.float32),
            pltpu.VMEM_SHARED((VPAD,), jnp.float32),
            pltpu.VMEM((SLOTS,), jnp.float32),
            pltpu.VMEM((SLOTS,), jnp.float32),
            pltpu.VMEM((SLOTS,), jnp.float32),
            pltpu.SemaphoreType.DMA,
            pltpu.SemaphoreType.DMA,
            pltpu.SemaphoreType.DMA,
        ],
        compiler_params=pltpu.CompilerParams(
            needs_layout_passes=False, use_tc_tiling_on_sc=True),
    )(_sc_body)
    return f(qid, did, tabt, tail)


def kernel(input_ids, attention_mask, token_type_ids, qid, did,
           session_qid, session_did, session_len, id2emb):
    qid3 = qid.astype(jnp.int32).reshape(NS, NCH, 128)
    did3 = did.astype(jnp.int32).reshape(NS, NCH, 128)
    tabt = id2emb.T
    tail = jnp.pad(tabt[:, TAIL_OFF:], ((0, 0), (0, TAIL - (VOCAB - TAIL_OFF))))
    partial = _run(qid3, did3, tabt, tail)
    return partial[0] + partial[1]


# speedup vs baseline: 1.1571x; 1.1571x over previous
import functools

import jax
import jax.numpy as jnp
from jax import lax
from jax.experimental import pallas as pl
from jax.experimental.pallas import tpu as pltpu
from jax.experimental.pallas import tpu_sc as plsc

VOCAB = 1000000
EMB = 64
BATCH = 16384
NC = 2
NS = 16
L = 16
SLOTS = BATCH // NS
D_PER_CORE = EMB // NC
NPAIR = D_PER_CORE // 2
NCH = SLOTS // 128
RCH = (VOCAB // NS) // 128 * 128
TAIL_OFF = RCH * NS
TAIL = 640
VPAD = TAIL_OFF + TAIL


def _sc_body(qid_hbm, did_hbm, tabt_hbm, tail_hbm, out_hbm,
             qid_v, did_v, buf0, buf1, qv_v, dv_v, acc_v,
             sem0, sem1, semg):
    c = lax.axis_index("c")
    s = lax.axis_index("s")
    d_base = c * D_PER_CORE

    pltpu.sync_copy(qid_hbm.at[s], qid_v)
    pltpu.sync_copy(did_hbm.at[s], did_v)

    zero = jnp.zeros((L,), jnp.float32)
    for k in range(SLOTS // L):
        acc_v[pl.ds(k * L, L)] = zero

    def row_copies(d, buf, sem):
        sl = pl.ds(s * RCH, RCH)
        tl = pl.ds(TAIL_OFF, TAIL)
        return (pltpu.make_async_copy(tabt_hbm.at[d].at[sl], buf.at[sl], sem),
                pltpu.make_async_copy(tail_hbm.at[d], buf.at[tl], sem))

    def row_start(d, buf, sem):
        main, tail = row_copies(d, buf, sem)
        main.start()

        @pl.when(s == 0)
        def _():
            tail.start()

    def row_wait(d, buf, sem):
        main, tail = row_copies(d, buf, sem)
        main.wait()

        @pl.when(s == 0)
        def _():
            tail.wait()

    def process(buf):
        copies = []
        for j in range(NCH):
            copies.append(pltpu.async_copy(
                buf.at[qid_v.at[j]], qv_v.at[pl.ds(j * 128, 128)], semg))
            copies.append(pltpu.async_copy(
                buf.at[did_v.at[j]], dv_v.at[pl.ds(j * 128, 128)], semg))
        for cp in copies:
            cp.wait()
        for k in range(SLOTS // L):
            sl = pl.ds(k * L, L)
            plsc.addupdate(acc_v.at[sl], qv_v[sl] * dv_v[sl])

    row_start(d_base, buf0, sem0)

    def pair(i, carry):
        d0 = d_base + i * 2

        row_start(d0 + 1, buf1, sem1)
        row_wait(d0, buf0, sem0)

        plsc.subcore_barrier()
        process(buf0)
        plsc.subcore_barrier()

        @pl.when(i + 1 < NPAIR)
        def _():
            row_start(d0 + 2, buf0, sem0)

        row_wait(d0 + 1, buf1, sem1)

        plsc.subcore_barrier()
        process(buf1)
        plsc.subcore_barrier()
        return carry

    lax.fori_loop(0, NPAIR, pair, 0)

    pltpu.sync_copy(acc_v, out_hbm.at[c, pl.ds(s * SLOTS, SLOTS)])


@jax.jit
def _run(qid, did, tabt, tail):
    mesh = plsc.VectorSubcoreMesh(core_axis_name="c", subcore_axis_name="s")
    f = functools.partial(
        pl.kernel, mesh=mesh,
        out_type=jax.ShapeDtypeStruct((NC, BATCH), jnp.float32),
        scratch_types=[
            pltpu.VMEM((NCH, 128), jnp.int32),
            pltpu.VMEM((NCH, 128), jnp.int32),
            pltpu.VMEM_SHARED((VPAD,), jnp.float32),
            pltpu.VMEM_SHARED((VPAD,), jnp.float32),
            pltpu.VMEM((SLOTS,), jnp.float32),
            pltpu.VMEM((SLOTS,), jnp.float32),
            pltpu.VMEM((SLOTS,), jnp.float32),
            pltpu.SemaphoreType.DMA,
            pltpu.SemaphoreType.DMA,
            pltpu.SemaphoreType.DMA,
        ],
        compiler_params=pltpu.CompilerParams(
            needs_layout_passes=False, use_tc_tiling_on_sc=True),
    )(_sc_body)
    return f(qid, did, tabt, tail)


def kernel(input_ids, attention_mask, token_type_ids, qid, did,
           session_qid, session_did, session_len, id2emb):
    qid3 = qid.astype(jnp.int32).reshape(NS, NCH, 128)
    did3 = did.astype(jnp.int32).reshape(NS, NCH, 128)
    tabt = id2emb.T
    tail = jnp.pad(tabt[:, TAIL_OFF:], ((0, 0), (0, TAIL - (VOCAB - TAIL_OFF))))
    partial = _run(qid3, did3, tabt, tail)
    return partial[0] + partial[1]


# speedup vs baseline: 1.1800x; 1.0198x over previous
import functools

import jax
import jax.numpy as jnp
from jax import lax
from jax.experimental import pallas as pl
from jax.experimental.pallas import tpu as pltpu
from jax.experimental.pallas import tpu_sc as plsc

VOCAB = 1000000
EMB = 64
BATCH = 16384
NC = 2
NS = 16
L = 16
SLOTS = BATCH // NS
D_PER_CORE = EMB // NC
NPAIR = D_PER_CORE // 2
NCH = SLOTS // 128
RCH = (VOCAB // NS) // 128 * 128
TAIL_OFF = RCH * NS
TAIL = 640
VPAD = TAIL_OFF + TAIL


def _sc_body(qid_hbm, did_hbm, tabt_hbm, tail_hbm, out_hbm,
             qid_v, did_v, buf0, buf1, qv_v, dv_v, acc_v,
             sem0, sem1, semg):
    c = lax.axis_index("c")
    s = lax.axis_index("s")
    d_base = c * D_PER_CORE

    pltpu.sync_copy(qid_hbm.at[s], qid_v)
    pltpu.sync_copy(did_hbm.at[s], did_v)

    zero = jnp.zeros((L,), jnp.float32)
    for k in range(SLOTS // L):
        acc_v[pl.ds(k * L, L)] = zero

    def row_copies(d, buf, sem):
        sl = pl.ds(s * RCH, RCH)
        tl = pl.ds(TAIL_OFF, TAIL)
        return (pltpu.make_async_copy(tabt_hbm.at[d].at[sl], buf.at[sl], sem),
                pltpu.make_async_copy(tail_hbm.at[d], buf.at[tl], sem))

    def row_start(d, buf, sem):
        main, tail = row_copies(d, buf, sem)
        main.start()

        @pl.when(s == 0)
        def _():
            tail.start()

    def row_wait(d, buf, sem):
        main, tail = row_copies(d, buf, sem)
        main.wait()

        @pl.when(s == 0)
        def _():
            tail.wait()

    def process(buf):
        copies = []
        for j in range(NCH):
            copies.append(pltpu.async_copy(
                buf.at[qid_v.at[j]], qv_v.at[pl.ds(j * 128, 128)], semg))
            copies.append(pltpu.async_copy(
                buf.at[did_v.at[j]], dv_v.at[pl.ds(j * 128, 128)], semg))
        for j in range(NCH):
            copies[2 * j].wait()
            copies[2 * j + 1].wait()
            for k in range(128 // L):
                sl = pl.ds(j * 128 + k * L, L)
                plsc.addupdate(acc_v.at[sl], qv_v[sl] * dv_v[sl])

    row_start(d_base, buf0, sem0)

    def pair(i, carry):
        d0 = d_base + i * 2

        row_wait(d0, buf0, sem0)
        row_start(d0 + 1, buf1, sem1)

        plsc.subcore_barrier()
        process(buf0)
        plsc.subcore_barrier()

        row_wait(d0 + 1, buf1, sem1)

        @pl.when(i + 1 < NPAIR)
        def _():
            row_start(d0 + 2, buf0, sem0)

        plsc.subcore_barrier()
        process(buf1)
        plsc.subcore_barrier()
        return carry

    lax.fori_loop(0, NPAIR, pair, 0)

    pltpu.sync_copy(acc_v, out_hbm.at[c, pl.ds(s * SLOTS, SLOTS)])


@jax.jit
def _run(qid, did, tabt, tail):
    mesh = plsc.VectorSubcoreMesh(core_axis_name="c", subcore_axis_name="s")
    f = functools.partial(
        pl.kernel, mesh=mesh,
        out_type=jax.ShapeDtypeStruct((NC, BATCH), jnp.float32),
        scratch_types=[
            pltpu.VMEM((NCH, 128), jnp.int32),
            pltpu.VMEM((NCH, 128), jnp.int32),
            pltpu.VMEM_SHARED((VPAD,), jnp.float32),
            pltpu.VMEM_SHARED((VPAD,), jnp.float32),
            pltpu.VMEM((SLOTS,), jnp.float32),
            pltpu.VMEM((SLOTS,), jnp.float32),
            pltpu.VMEM((SLOTS,), jnp.float32),
            pltpu.SemaphoreType.DMA,
            pltpu.SemaphoreType.DMA,
            pltpu.SemaphoreType.DMA,
        ],
        compiler_params=pltpu.CompilerParams(
            needs_layout_passes=False, use_tc_tiling_on_sc=True),
    )(_sc_body)
    return f(qid, did, tabt, tail)


def kernel(input_ids, attention_mask, token_type_ids, qid, did,
           session_qid, session_did, session_len, id2emb):
    qid3 = qid.astype(jnp.int32).reshape(NS, NCH, 128)
    did3 = did.astype(jnp.int32).reshape(NS, NCH, 128)
    tabt = id2emb.T
    tail = jnp.pad(tabt[:, TAIL_OFF:], ((0, 0), (0, TAIL - (VOCAB - TAIL_OFF))))
    partial = _run(qid3, did3, tabt, tail)
    return partial[0] + partial[1]


# speedup vs baseline: 1.1817x; 1.0014x over previous
import functools

import jax
import jax.numpy as jnp
from jax import lax
from jax.experimental import pallas as pl
from jax.experimental.pallas import tpu as pltpu
from jax.experimental.pallas import tpu_sc as plsc

VOCAB = 1000000
EMB = 64
BATCH = 16384
NC = 2
NS = 16
L = 16
SLOTS = BATCH // NS
D_PER_CORE = EMB // NC
NPAIR = D_PER_CORE // 2
NCH = SLOTS // 128
RCH = (VOCAB // NS) // 128 * 128
TAIL_OFF = RCH * NS
TAIL = 640
VPAD = TAIL_OFF + TAIL


def _sc_body(qid_hbm, did_hbm, tabt_hbm, tail_hbm, out_hbm,
             qid_v, did_v, buf0, buf1, qv_v, dv_v, acc_v,
             sem0, sem1, semg):
    c = lax.axis_index("c")
    s = lax.axis_index("s")
    d_base = c * D_PER_CORE

    pltpu.sync_copy(qid_hbm.at[s], qid_v)
    pltpu.sync_copy(did_hbm.at[s], did_v)

    zero = jnp.zeros((L,), jnp.float32)
    for k in range(SLOTS // L):
        acc_v[pl.ds(k * L, L)] = zero

    HCH = RCH // 2

    def row_copies(d, buf, sem):
        sla = pl.ds(s * RCH, HCH)
        slb = pl.ds(s * RCH + HCH, HCH)
        tl = pl.ds(TAIL_OFF, TAIL)
        return (pltpu.make_async_copy(tabt_hbm.at[d].at[sla], buf.at[sla], sem),
                pltpu.make_async_copy(tabt_hbm.at[d].at[slb], buf.at[slb], sem),
                pltpu.make_async_copy(tail_hbm.at[d], buf.at[tl], sem))

    def row_start(d, buf, sem):
        ca, cb, tail = row_copies(d, buf, sem)
        ca.start()
        cb.start()

        @pl.when(s == 0)
        def _():
            tail.start()

    def row_wait(d, buf, sem):
        ca, cb, tail = row_copies(d, buf, sem)
        ca.wait()
        cb.wait()

        @pl.when(s == 0)
        def _():
            tail.wait()

    def process(buf):
        copies = []
        for j in range(NCH):
            copies.append(pltpu.async_copy(
                buf.at[qid_v.at[j]], qv_v.at[pl.ds(j * 128, 128)], semg))
            copies.append(pltpu.async_copy(
                buf.at[did_v.at[j]], dv_v.at[pl.ds(j * 128, 128)], semg))
        for j in range(NCH):
            copies[2 * j].wait()
            copies[2 * j + 1].wait()
            for k in range(128 // L):
                sl = pl.ds(j * 128 + k * L, L)
                plsc.addupdate(acc_v.at[sl], qv_v[sl] * dv_v[sl])

    row_start(d_base, buf0, sem0)

    def pair(i, carry):
        d0 = d_base + i * 2

        row_wait(d0, buf0, sem0)
        row_start(d0 + 1, buf1, sem1)

        plsc.subcore_barrier()
        process(buf0)
        plsc.subcore_barrier()

        row_wait(d0 + 1, buf1, sem1)

        @pl.when(i + 1 < NPAIR)
        def _():
            row_start(d0 + 2, buf0, sem0)

        plsc.subcore_barrier()
        process(buf1)
        plsc.subcore_barrier()
        return carry

    lax.fori_loop(0, NPAIR, pair, 0)

    pltpu.sync_copy(acc_v, out_hbm.at[c, pl.ds(s * SLOTS, SLOTS)])


@jax.jit
def _run(qid, did, tabt, tail):
    mesh = plsc.VectorSubcoreMesh(core_axis_name="c", subcore_axis_name="s")
    f = functools.partial(
        pl.kernel, mesh=mesh,
        out_type=jax.ShapeDtypeStruct((NC, BATCH), jnp.float32),
        scratch_types=[
            pltpu.VMEM((NCH, 128), jnp.int32),
            pltpu.VMEM((NCH, 128), jnp.int32),
            pltpu.VMEM_SHARED((VPAD,), jnp.float32),
            pltpu.VMEM_SHARED((VPAD,), jnp.float32),
            pltpu.VMEM((SLOTS,), jnp.float32),
            pltpu.VMEM((SLOTS,), jnp.float32),
            pltpu.VMEM((SLOTS,), jnp.float32),
            pltpu.SemaphoreType.DMA,
            pltpu.SemaphoreType.DMA,
            pltpu.SemaphoreType.DMA,
        ],
        compiler_params=pltpu.CompilerParams(
            needs_layout_passes=False, use_tc_tiling_on_sc=True),
    )(_sc_body)
    return f(qid, did, tabt, tail)


def kernel(input_ids, attention_mask, token_type_ids, qid, did,
           session_qid, session_did, session_len, id2emb):
    qid3 = qid.astype(jnp.int32).reshape(NS, NCH, 128)
    did3 = did.astype(jnp.int32).reshape(NS, NCH, 128)
    tabt = id2emb.T
    tail = jnp.pad(tabt[:, TAIL_OFF:], ((0, 0), (0, TAIL - (VOCAB - TAIL_OFF))))
    partial = _run(qid3, did3, tabt, tail)
    return partial[0] + partial[1]
